# baseline (speedup 1.0000x reference)
import functools

import jax
import jax.numpy as jnp
from jax import lax
from jax.experimental import pallas as pl
from jax.experimental.pallas import tpu as pltpu
from jax.experimental.pallas import tpu_sc as plsc


@functools.cache
def _make_detranspose(V, D):
    info = plsc.get_sparse_core_info()
    nc, ns = info.num_cores, info.num_subcores
    NW = nc * ns
    NB_FULL = V // 128
    TAIL = (V - NB_FULL * 128) * D
    PER_W = (NB_FULL + NW - 1) // NW
    BLK = 128 * D
    mesh = plsc.VectorSubcoreMesh(core_axis_name="c", subcore_axis_name="s")

    @functools.partial(
        pl.kernel,
        mesh=mesh,
        out_type=jax.ShapeDtypeStruct((V * D,), jnp.float32),
        scratch_types=[
            pltpu.VMEM((2, D, 137), jnp.float32),
            pltpu.VMEM((2, BLK), jnp.float32),
            pltpu.VMEM((max(TAIL, 8),), jnp.float32),
            pltpu.SemaphoreType.DMA,
            pltpu.SemaphoreType.DMA,
            pltpu.SemaphoreType.DMA,
        ],
        compiler_params=pltpu.CompilerParams(
            use_tc_tiling_on_sc=True, needs_layout_passes=False
        ),
    )
    def k(wt_hbm, tail_hbm, out_hbm, in_v, cmp_v, tail_v, isem, osem, tsem):
        wid = lax.axis_index("s") * nc + lax.axis_index("c")

        def c0_of(kk):
            return jnp.minimum(wid * PER_W + kk, NB_FULL - 1) * 128

        def in_dst(buf):
            return in_v.at[buf, :, pl.ds(0, 128)]

        pltpu.async_copy(wt_hbm.at[:, pl.ds(c0_of(0), 128)], in_dst(0), isem)

        if TAIL:
            @pl.when(wid == NW - 1)
            def _():
                pltpu.async_copy(tail_hbm, tail_v.at[pl.ds(0, TAIL)], tsem).wait()
                pltpu.async_copy(
                    tail_v.at[pl.ds(0, TAIL)],
                    out_hbm.at[pl.ds(NB_FULL * BLK, TAIL)],
                    tsem,
                ).wait()

        def per_blk(kk, carry):
            buf = lax.rem(kk, 2)
            nxt = 1 - buf

            @pl.when(kk >= 2)
            def _():
                pltpu.make_async_copy(
                    cmp_v.at[buf], out_hbm.at[pl.ds(0, BLK)], osem
                ).wait()

            pltpu.make_async_copy(
                wt_hbm.at[:, pl.ds(c0_of(kk), 128)], in_dst(buf), isem
            ).wait()

            @pl.when(kk + 1 < PER_W)
            def _():
                pltpu.async_copy(
                    wt_hbm.at[:, pl.ds(c0_of(kk + 1), 128)], in_dst(nxt), isem
                )

            def tbody(i, c2):
                for u in range(4):
                    ii = i * 4 + u
                    i_idx = jnp.full((16,), ii, jnp.int32)
                    for j in range(D // 16):
                        d_idx = lax.iota(jnp.int32, 16) + (j * 16)
                        x = plsc.load_gather(in_v.at[buf], [d_idx, i_idx])
                        cmp_v[buf, pl.ds(ii * D + j * 16, 16)] = x
                return c2

            lax.fori_loop(0, 32, tbody, 0)

            pltpu.async_copy(
                cmp_v.at[buf], out_hbm.at[pl.ds(c0_of(kk) * D, BLK)], osem
            )
            return carry

        lax.fori_loop(0, PER_W, per_blk, 0)

        for _ in range(2):
            pltpu.make_async_copy(
                cmp_v.at[0], out_hbm.at[pl.ds(0, BLK)], osem
            ).wait()

    return k


@functools.cache
def _make_embed(T, B, D, V):
    info = plsc.get_sparse_core_info()
    nc, ns = info.num_cores, info.num_subcores
    NW = nc * ns
    CH = B // NW
    NCB = CH // 128
    NR = D // 8
    SKEW = CH + 9
    mesh = plsc.VectorSubcoreMesh(core_axis_name="c", subcore_axis_name="s")

    @functools.partial(
        pl.kernel,
        mesh=mesh,
        out_type=jax.ShapeDtypeStruct((T, NR, B // 128, 8, 128), jnp.float32),
        scratch_types=[
            pltpu.VMEM((2, CH), jnp.int32),
            pltpu.VMEM((2, CH, D), jnp.float32),
            pltpu.VMEM((2, D, SKEW), jnp.float32),
            pltpu.SemaphoreType.DMA,
            pltpu.SemaphoreType.DMA,
            pltpu.SemaphoreType.DMA,
        ],
        compiler_params=pltpu.CompilerParams(
            use_tc_tiling_on_sc=False, needs_layout_passes=False
        ),
    )
    def k(idx_hbm, table_hbm, out_hbm, idx_v, rows_v, tr_v, isem, gsem, osem):
        wid = lax.axis_index("s") * nc + lax.axis_index("c")
        base_b = wid * CH

        def idx_src(t):
            return idx_hbm.at[pl.ds(t * B + base_b, CH)]

        pltpu.async_copy(idx_src(0), idx_v.at[0], isem)

        def per_t(t, carry):
            b = lax.rem(t, 2)
            p = 1 - b

            @pl.when(t >= 2)
            def _():
                for r in range(NR):
                    for c in range(NCB):
                        pltpu.make_async_copy(
                            tr_v.at[b, pl.ds(r * 8, 8), pl.ds(c * 128, 128)],
                            out_hbm.at[0, r, wid * NCB + c],
                            osem,
                        ).wait()

            @pl.when(t < T)
            def _():
                pltpu.make_async_copy(idx_src(t), idx_v.at[b], isem).wait()
                pltpu.async_copy(table_hbm.at[idx_v.at[b]], rows_v.at[b], gsem)

            @pl.when(t >= 1)
            def _():
                pltpu.make_async_copy(
                    table_hbm.at[idx_v.at[p]], rows_v.at[p], gsem
                ).wait()

                def tbody(i, c2):
                    for u in range(8):
                        bb = i * 8 + u
                        b_idx = jnp.full((16,), bb, jnp.int32)
                        for j in range(D // 16):
                            x = rows_v[p, bb, pl.ds(j * 16, 16)]
                            d_idx = lax.iota(jnp.int32, 16) + (j * 16)
                            plsc.store_scatter(tr_v.at[p], [d_idx, b_idx], x)
                    return c2

                lax.fori_loop(0, CH // 8, tbody, 0)

                for r in range(NR):
                    for c in range(NCB):
                        pltpu.async_copy(
                            tr_v.at[p, pl.ds(r * 8, 8), pl.ds(c * 128, 128)],
                            out_hbm.at[t - 1, r, wid * NCB + c],
                            osem,
                        )

            @pl.when(t + 1 < T)
            def _():
                pltpu.async_copy(idx_src(t + 1), idx_v.at[p], isem)

            return carry

        lax.fori_loop(0, T + 1, per_t, 0)

        for r in range(NR):
            for c in range(NCB):
                pltpu.make_async_copy(
                    tr_v.at[0, pl.ds(r * 8, 8), pl.ds(c * 128, 128)],
                    out_hbm.at[0, r, wid * NCB + c],
                    osem,
                ).wait()

    return k


def kernel(X, W):
    B, T = X.shape
    V, D = W.shape
    nfull = (V // 128) * 128
    wt = jnp.transpose(W)
    tail = lax.slice(W, (nfull, 0), (V, D)).reshape((V - nfull) * D)
    wlin = _make_detranspose(V, D)(wt, tail)
    w2 = wlin.reshape(V, D)
    idx = X.transpose(1, 0).reshape(T * B).astype(jnp.int32)
    out5 = _make_embed(T, B, D, V)(idx, w2)
    return out5.transpose(2, 4, 0, 1, 3).reshape(B, T, D)


# speedup vs baseline: 1.9574x; 1.9574x over previous
import functools

import jax
import jax.numpy as jnp
from jax import lax
from jax.experimental import pallas as pl
from jax.experimental.pallas import tpu as pltpu
from jax.experimental.pallas import tpu_sc as plsc


@functools.cache
def _make_detranspose(V, D):
    info = plsc.get_sparse_core_info()
    nc, ns = info.num_cores, info.num_subcores
    NW = nc * ns
    NB_FULL = V // 128
    TAIL = (V - NB_FULL * 128) * D
    PER_W = (NB_FULL + NW - 1) // NW
    BLK = 128 * D
    mesh = plsc.VectorSubcoreMesh(core_axis_name="c", subcore_axis_name="s")

    @functools.partial(
        pl.kernel,
        mesh=mesh,
        out_type=jax.ShapeDtypeStruct((V * D,), jnp.float32),
        scratch_types=[
            pltpu.VMEM((4, D, 137), jnp.float32),
            pltpu.VMEM((4, BLK), jnp.float32),
            pltpu.VMEM((max(TAIL, 8),), jnp.float32),
            pltpu.SemaphoreType.DMA,
            pltpu.SemaphoreType.DMA,
            pltpu.SemaphoreType.DMA,
        ],
        compiler_params=pltpu.CompilerParams(
            use_tc_tiling_on_sc=True, needs_layout_passes=False
        ),
    )
    def k(wt_hbm, tail_hbm, out_hbm, in_v, cmp_v, tail_v, isem, osem, tsem):
        wid = lax.axis_index("s") * nc + lax.axis_index("c")

        def c0_of(kk):
            return jnp.minimum(wid * PER_W + kk, NB_FULL - 1) * 128

        def in_dst(buf):
            return in_v.at[buf, :, pl.ds(0, 128)]

        for pre in range(3):
            pltpu.async_copy(
                wt_hbm.at[:, pl.ds(c0_of(pre), 128)], in_dst(pre), isem
            )

        if TAIL:
            @pl.when(wid == NW - 1)
            def _():
                pltpu.async_copy(tail_hbm, tail_v.at[pl.ds(0, TAIL)], tsem).wait()
                pltpu.async_copy(
                    tail_v.at[pl.ds(0, TAIL)],
                    out_hbm.at[pl.ds(NB_FULL * BLK, TAIL)],
                    tsem,
                ).wait()

        def per_blk(kk, carry):
            buf = lax.rem(kk, 4)

            @pl.when(kk >= 3)
            def _():
                pltpu.make_async_copy(
                    cmp_v.at[buf], out_hbm.at[pl.ds(0, BLK)], osem
                ).wait()

            pltpu.make_async_copy(
                wt_hbm.at[:, pl.ds(c0_of(kk), 128)], in_dst(buf), isem
            ).wait()

            @pl.when(kk + 3 < PER_W)
            def _():
                pltpu.async_copy(
                    wt_hbm.at[:, pl.ds(c0_of(kk + 3), 128)],
                    in_dst(lax.rem(kk + 3, 4)),
                    isem,
                )

            @plsc.parallel_loop(0, 128, unroll=8)
            def _(ii):
                i_idx = jnp.full((16,), ii, jnp.int32)
                for j in range(D // 16):
                    d_idx = lax.iota(jnp.int32, 16) + (j * 16)
                    x = plsc.load_gather(in_v.at[buf], [d_idx, i_idx])
                    cmp_v[buf, pl.ds(ii * D + j * 16, 16)] = x

            pltpu.async_copy(
                cmp_v.at[buf], out_hbm.at[pl.ds(c0_of(kk) * D, BLK)], osem
            )
            return carry

        lax.fori_loop(0, PER_W, per_blk, 0)

        for _ in range(3):
            pltpu.make_async_copy(
                cmp_v.at[0], out_hbm.at[pl.ds(0, BLK)], osem
            ).wait()

    return k


@functools.cache
def _make_embed(T, B, D, V):
    info = plsc.get_sparse_core_info()
    nc, ns = info.num_cores, info.num_subcores
    NW = nc * ns
    CH = B // NW
    NCB = CH // 128
    NR = D // 8
    SKEW = CH + 9
    mesh = plsc.VectorSubcoreMesh(core_axis_name="c", subcore_axis_name="s")

    @functools.partial(
        pl.kernel,
        mesh=mesh,
        out_type=jax.ShapeDtypeStruct((T, NR, B // 128, 8, 128), jnp.float32),
        scratch_types=[
            pltpu.VMEM((2, CH), jnp.int32),
            pltpu.VMEM((2, CH, D), jnp.float32),
            pltpu.VMEM((2, D, SKEW), jnp.float32),
            pltpu.SemaphoreType.DMA,
            pltpu.SemaphoreType.DMA,
            pltpu.SemaphoreType.DMA,
        ],
        compiler_params=pltpu.CompilerParams(
            use_tc_tiling_on_sc=False, needs_layout_passes=False
        ),
    )
    def k(idx_hbm, table_hbm, out_hbm, idx_v, rows_v, tr_v, isem, gsem, osem):
        wid = lax.axis_index("s") * nc + lax.axis_index("c")
        base_b = wid * CH

        def idx_src(t):
            return idx_hbm.at[pl.ds(t * B + base_b, CH)]

        pltpu.async_copy(idx_src(0), idx_v.at[0], isem)

        def per_t(t, carry):
            b = lax.rem(t, 2)
            p = 1 - b

            @pl.when(t >= 2)
            def _():
                for r in range(NR):
                    for c in range(NCB):
                        pltpu.make_async_copy(
                            tr_v.at[b, pl.ds(r * 8, 8), pl.ds(c * 128, 128)],
                            out_hbm.at[0, r, wid * NCB + c],
                            osem,
                        ).wait()

            @pl.when(t < T)
            def _():
                pltpu.make_async_copy(idx_src(t), idx_v.at[b], isem).wait()
                pltpu.async_copy(table_hbm.at[idx_v.at[b]], rows_v.at[b], gsem)

            @pl.when(t >= 1)
            def _():
                pltpu.make_async_copy(
                    table_hbm.at[idx_v.at[p]], rows_v.at[p], gsem
                ).wait()

                @plsc.parallel_loop(0, CH, unroll=8)
                def _(bb):
                    b_idx = jnp.full((16,), bb, jnp.int32)
                    for j in range(D // 16):
                        x = rows_v[p, bb, pl.ds(j * 16, 16)]
                        d_idx = lax.iota(jnp.int32, 16) + (j * 16)
                        plsc.store_scatter(tr_v.at[p], [d_idx, b_idx], x)

                for r in range(NR):
                    for c in range(NCB):
                        pltpu.async_copy(
                            tr_v.at[p, pl.ds(r * 8, 8), pl.ds(c * 128, 128)],
                            out_hbm.at[t - 1, r, wid * NCB + c],
                            osem,
                        )

            @pl.when(t + 1 < T)
            def _():
                pltpu.async_copy(idx_src(t + 1), idx_v.at[p], isem)

            return carry

        lax.fori_loop(0, T + 1, per_t, 0)

        for r in range(NR):
            for c in range(NCB):
                pltpu.make_async_copy(
                    tr_v.at[0, pl.ds(r * 8, 8), pl.ds(c * 128, 128)],
                    out_hbm.at[0, r, wid * NCB + c],
                    osem,
                ).wait()

    return k


def kernel(X, W):
    B, T = X.shape
    V, D = W.shape
    nfull = (V // 128) * 128
    wt = jnp.transpose(W)
    tail = lax.slice(W, (nfull, 0), (V, D)).reshape((V - nfull) * D)
    wlin = _make_detranspose(V, D)(wt, tail)
    w2 = wlin.reshape(V, D)
    idx = X.transpose(1, 0).reshape(T * B).astype(jnp.int32)
    out5 = _make_embed(T, B, D, V)(idx, w2)
    return out5.transpose(2, 4, 0, 1, 3).reshape(B, T, D)
